# Initial kernel scaffold; baseline (speedup 1.0000x reference)
import jax
import jax.numpy as jnp
from jax.experimental import pallas as pl


def kernel(x, edge_index, W_gcn, b_gcn, w_ih, w_hh, b_ih, b_hh, W_edge, b_edge):
    raise NotImplementedError("write your pallas kernel here")


# speedup vs baseline: 11.4256x; 11.4256x over previous
import functools

import jax
import jax.numpy as jnp
from jax import lax
from jax.experimental import pallas as pl
from jax.experimental.pallas import tpu as pltpu
from jax.experimental.pallas import tpu_sc as plsc

NN = 10000
EE = 320000
DD = 128
HH = 128

NC, NS, LL = 2, 16, 16
NW = NC * NS
CH = 128
EPW_RAW = EE // NW
NCH = (EPW_RAW + CH - 1) // CH
EPW = NCH * CH
NT = EPW
RPT = NT // NS


def _mm_body(x_ref, w_ref, o_ref):
    o_ref[...] = jnp.dot(x_ref[...], w_ref[...],
                         preferred_element_type=jnp.float32)


def _tc_matmul(x, w):
    return pl.pallas_call(
        _mm_body,
        out_shape=jax.ShapeDtypeStruct((x.shape[0], w.shape[1]), jnp.float32),
    )(x, w)


def _prep_body(deg_ref, xw_ref, dinv_ref, u_ref):
    deg = deg_ref[0, :] + deg_ref[1, :] + 1.0
    dinv = lax.rsqrt(deg)
    dinv_ref[...] = dinv[:, None]
    u_ref[...] = dinv[:, None] * xw_ref[...]


def _tc_prep(degcols, xw):
    return pl.pallas_call(
        _prep_body,
        out_shape=(
            jax.ShapeDtypeStruct((NN, 1), jnp.float32),
            jax.ShapeDtypeStruct((NN, DD), jnp.float32),
        ),
    )(degcols, xw)


def _gi_body(s0_ref, s1_ref, xw_ref, dinv_ref, bg_ref, wih_ref, bih_ref,
             gi_ref):
    dinv = dinv_ref[...]
    agg = dinv * (s0_ref[...] + s1_ref[...]) \
        + (dinv * dinv) * xw_ref[...] + bg_ref[...]
    xg = jnp.maximum(agg, 0.0)
    gi_ref[...] = lax.dot_general(
        xg, wih_ref[...], (((1,), (1,)), ((), ())),
        preferred_element_type=jnp.float32) + bih_ref[...]


def _tc_gi(s0, s1, xw, dinv, b_gcn, w_ih, b_ih):
    return pl.pallas_call(
        _gi_body,
        out_shape=jax.ShapeDtypeStruct((NN, 3 * HH), jnp.float32),
    )(s0, s1, xw, dinv, b_gcn, w_ih, b_ih)


def _gru_body(gi_ref, whh_ref, bhh_ref, we_ref, be_ref, y_ref, hl_ref,
              seq_ref):
    whh = whh_ref[...]
    bhh = bhh_ref[...]

    def step(t, h):
        gi = gi_ref[pl.ds(t, 1), :]
        gh = lax.dot_general(h, whh, (((1,), (1,)), ((), ())),
                             preferred_element_type=jnp.float32) + bhh
        r = jax.nn.sigmoid(gi[:, 0:HH] + gh[:, 0:HH])
        z = jax.nn.sigmoid(gi[:, HH:2 * HH] + gh[:, HH:2 * HH])
        n = jnp.tanh(gi[:, 2 * HH:] + r * gh[:, 2 * HH:])
        h2 = (1.0 - z) * n + z * h
        seq_ref[pl.ds(t, 1), :] = h2
        return h2

    h_last = lax.fori_loop(0, NN, step, jnp.zeros((1, HH), jnp.float32))
    hl_ref[...] = h_last
    y_ref[...] = 0.5 * (jnp.dot(seq_ref[...], we_ref[...],
                                preferred_element_type=jnp.float32)
                        + be_ref[...])


def _tc_gru(gi, w_hh, b_hh, w_edge, b_edge):
    return pl.pallas_call(
        _gru_body,
        out_shape=(
            jax.ShapeDtypeStruct((NN, HH), jnp.float32),
            jax.ShapeDtypeStruct((1, HH), jnp.float32),
        ),
        scratch_shapes=[pltpu.VMEM((NN, HH), jnp.float32)],
    )(gi, w_hh, b_hh, w_edge, b_edge)


def _sc_deg_body(dst_hbm, zeros_hbm, ones_hbm, out_hbm, idx_v, ones_v, shared):
    c = lax.axis_index("c")
    s = lax.axis_index("s")
    wid = c * NS + s
    pltpu.sync_copy(zeros_hbm, shared.at[pl.ds(s * RPT, RPT)])
    pltpu.sync_copy(dst_hbm.at[wid], idx_v)
    pltpu.sync_copy(ones_hbm, ones_v)
    plsc.subcore_barrier()

    def body(j, carry):
        pltpu.sync_copy(ones_v, shared.at[idx_v.at[j]], add=True)
        return carry

    lax.fori_loop(0, NCH, body, 0)
    plsc.subcore_barrier()
    pltpu.sync_copy(shared.at[pl.ds(s * RPT, RPT)],
                    out_hbm.at[c, pl.ds(s * RPT, RPT)])


def _sc_scatter_body(u_hbm, src_hbm, dst_hbm, zeros_hbm, out_hbm,
                     idxr_v, idxc_v, rows_v, sem, shared):
    c = lax.axis_index("c")
    s = lax.axis_index("s")
    wid = c * NS + s
    pltpu.sync_copy(zeros_hbm, shared.at[pl.ds(s * RPT, RPT)])
    pltpu.sync_copy(src_hbm.at[wid], idxr_v)
    pltpu.sync_copy(dst_hbm.at[wid], idxc_v)
    plsc.subcore_barrier()

    def body(j, carry):
        pltpu.async_copy(u_hbm.at[idxr_v.at[j]], rows_v, sem).wait()
        pltpu.sync_copy(rows_v, shared.at[idxc_v.at[j]], add=True)
        return carry

    lax.fori_loop(0, NCH, body, 0)
    plsc.subcore_barrier()
    pltpu.sync_copy(shared.at[pl.ds(s * RPT, RPT)],
                    out_hbm.at[c, pl.ds(s * RPT, RPT)])


def _sc_edge_body(y_hbm, src_hbm, dst_hbm, out_hbm,
                  idxr_v, idxc_v, ra_v, rb_v, sema, semb):
    c = lax.axis_index("c")
    s = lax.axis_index("s")
    wid = c * NS + s
    pltpu.sync_copy(src_hbm.at[wid], idxr_v)
    pltpu.sync_copy(dst_hbm.at[wid], idxc_v)

    def body(j, carry):
        cp_a = pltpu.async_copy(y_hbm.at[idxr_v.at[j]], ra_v, sema)
        cp_b = pltpu.async_copy(y_hbm.at[idxc_v.at[j]], rb_v, semb)
        cp_a.wait()
        cp_b.wait()

        def row(r, carry2):
            for k in range(HH // LL):
                sl = pl.ds(k * LL, LL)
                ra_v[r, sl] = ra_v[r, sl] + rb_v[r, sl]
            return carry2

        lax.fori_loop(0, CH, row, 0)
        pltpu.sync_copy(ra_v, out_hbm.at[wid, pl.ds(j * CH, CH)])
        return carry

    lax.fori_loop(0, NCH, body, 0)


@functools.lru_cache(maxsize=1)
def _sc_kernels():
    mesh = plsc.VectorSubcoreMesh(core_axis_name="c", subcore_axis_name="s",
                                  num_cores=NC, num_subcores=NS)
    sc_deg = pl.kernel(
        _sc_deg_body,
        out_type=jax.ShapeDtypeStruct((NC, NT, DD), jnp.float32),
        mesh=mesh,
        scratch_types=[
            pltpu.VMEM((NCH, CH), jnp.int32),
            pltpu.VMEM((CH, DD), jnp.float32),
            pltpu.VMEM_SHARED((NT, DD), jnp.float32),
        ],
    )
    sc_scatter = pl.kernel(
        _sc_scatter_body,
        out_type=jax.ShapeDtypeStruct((NC, NT, DD), jnp.float32),
        mesh=mesh,
        scratch_types=[
            pltpu.VMEM((NCH, CH), jnp.int32),
            pltpu.VMEM((NCH, CH), jnp.int32),
            pltpu.VMEM((CH, DD), jnp.float32),
            pltpu.SemaphoreType.DMA,
            pltpu.VMEM_SHARED((NT, DD), jnp.float32),
        ],
    )
    sc_edge = pl.kernel(
        _sc_edge_body,
        out_type=jax.ShapeDtypeStruct((NW, EPW, HH), jnp.float32),
        mesh=mesh,
        scratch_types=[
            pltpu.VMEM((NCH, CH), jnp.int32),
            pltpu.VMEM((NCH, CH), jnp.int32),
            pltpu.VMEM((CH, HH), jnp.float32),
            pltpu.VMEM((CH, HH), jnp.float32),
            pltpu.SemaphoreType.DMA,
            pltpu.SemaphoreType.DMA,
        ],
    )
    return sc_deg, sc_scatter, sc_edge


def kernel(x, edge_index, W_gcn, b_gcn, w_ih, w_hh, b_ih, b_hh,
           W_edge, b_edge):
    src = edge_index[0].reshape(NW, EPW_RAW)
    dst = edge_index[1].reshape(NW, EPW_RAW)
    pad = ((0, 0), (0, EPW - EPW_RAW))
    srcp = jnp.pad(src, pad, constant_values=NN).reshape(NW, NCH, CH)
    dstp = jnp.pad(dst, pad, constant_values=NN).reshape(NW, NCH, CH)

    ones128 = jnp.ones((CH, DD), jnp.float32)
    zeros128 = jnp.zeros((RPT, DD), jnp.float32)
    sc_deg, sc_scatter, sc_edge = _sc_kernels()

    xw = _tc_matmul(x, W_gcn)
    degp = sc_deg(dstp, zeros128, ones128)
    dinv, u = _tc_prep(degp[:, :NN, 0], xw)
    u_ext = jnp.pad(u, ((0, NT - NN), (0, 0)))
    sums = sc_scatter(u_ext, srcp, dstp, zeros128)
    gi = _tc_gi(sums[0, :NN], sums[1, :NN], xw, dinv,
                b_gcn.reshape(1, HH), w_ih, b_ih.reshape(1, 3 * HH))
    y, h_last = _tc_gru(gi, w_hh, b_hh.reshape(1, 3 * HH),
                        W_edge, b_edge.reshape(1, HH))
    y_ext = jnp.pad(y, ((0, NT - NN), (0, 0)))
    eoutp = sc_edge(y_ext, srcp, dstp)
    edge_output = eoutp[:, :EPW_RAW].reshape(EE, HH)
    hidden = h_last.reshape(1, 1, HH)
    return (edge_output, hidden)
